# baseline (speedup 1.0000x reference)
import functools

import jax
import jax.numpy as jnp
from jax import lax
from jax.experimental import pallas as pl
from jax.experimental.pallas import tpu as pltpu
from jax.experimental.pallas import tpu_sc as plsc

_ROWS = 16384
_COLS = 2048
_NC = 2
_NS = 16
_NW = _NC * _NS
_WROWS = _ROWS // _NW
_CR = 32
_NCHUNK = _WROWS // _CR

_mesh = plsc.VectorSubcoreMesh(core_axis_name="c", subcore_axis_name="s")


@functools.partial(
    pl.kernel,
    out_type=jax.ShapeDtypeStruct((_ROWS, _COLS), jnp.float32),
    mesh=_mesh,
    scratch_types=[pltpu.VMEM((_CR, _COLS), jnp.float32)],
)
def _sc_copy(x_hbm, o_hbm, buf):
    wid = lax.axis_index("s") * _NC + lax.axis_index("c")
    base = wid * _WROWS

    def body(i, carry):
        off = base + i * _CR
        pltpu.sync_copy(x_hbm.at[pl.ds(off, _CR)], buf)
        pltpu.sync_copy(buf, o_hbm.at[pl.ds(off, _CR)])
        return carry

    lax.fori_loop(0, _NCHUNK, body, 0)


def kernel(x):
    out = _sc_copy(x.reshape(_ROWS, _COLS))
    return out.reshape(x.shape)


# speedup vs baseline: 1.0701x; 1.0701x over previous
import functools

import jax
import jax.numpy as jnp
from jax import lax
from jax.experimental import pallas as pl
from jax.experimental.pallas import tpu as pltpu
from jax.experimental.pallas import tpu_sc as plsc

_ROWS = 16384
_COLS = 2048
_NC = 2
_NS = 16
_NW = _NC * _NS
_WROWS = _ROWS // _NW
_CR = 16
_NCHUNK = _WROWS // _CR

_mesh = plsc.VectorSubcoreMesh(core_axis_name="c", subcore_axis_name="s")


@functools.partial(
    pl.kernel,
    out_type=jax.ShapeDtypeStruct((_ROWS, _COLS), jnp.float32),
    mesh=_mesh,
    scratch_types=[
        pltpu.VMEM((_CR, _COLS), jnp.float32),
        pltpu.VMEM((_CR, _COLS), jnp.float32),
        pltpu.SemaphoreType.DMA((2,)),
        pltpu.SemaphoreType.DMA((2,)),
    ],
)
def _sc_copy(x_hbm, o_hbm, buf0, buf1, sem_in, sem_out):
    wid = lax.axis_index("s") * _NC + lax.axis_index("c")
    base = wid * _WROWS
    bufs = (buf0, buf1)

    def in_copy(j):
        b = j % 2
        sl = pl.ds(base + j * _CR, _CR)
        return pltpu.make_async_copy(x_hbm.at[sl], bufs[b], sem_in.at[b])

    def out_copy(j):
        b = j % 2
        sl = pl.ds(base + j * _CR, _CR)
        return pltpu.make_async_copy(bufs[b], o_hbm.at[sl], sem_out.at[b])

    in_copy(0).start()
    in_copy(1).start()
    for j in range(_NCHUNK):
        in_copy(j).wait()
        out_copy(j).start()
        if j + 2 < _NCHUNK:
            out_copy(j).wait()
            in_copy(j + 2).start()
    out_copy(_NCHUNK - 2).wait()
    out_copy(_NCHUNK - 1).wait()


def kernel(x):
    out = _sc_copy(x.reshape(_ROWS, _COLS))
    return out.reshape(x.shape)
